# Initial kernel scaffold; baseline (speedup 1.0000x reference)
import jax
import jax.numpy as jnp
from jax.experimental import pallas as pl


def kernel(context_words, target_word, negative_words, W_in, W_out):
    raise NotImplementedError("write your pallas kernel here")


# speedup vs baseline: 5.6124x; 5.6124x over previous
import functools

import jax
import jax.numpy as jnp
from jax import lax
from jax.experimental import pallas as pl
from jax.experimental.pallas import tpu as pltpu
from jax.experimental.pallas import tpu_sc as plsc

VOCAB = 100000
EMBED = 64
B = 4096
CTX = 20
NEG = 50
NPAIR = NEG + 1

NC = 2
NS = 16
L = 16
NW = NC * NS
ROWS_W = B // NW
BC = L
NCHUNK = ROWS_W // BC
NCHUNK_TOT = B // BC

_CTX_N = BC * CTX
_PAIR_N = BC * NPAIR

def _dma_slices(total, step=128):
    out, off = [], 0
    while off < total:
        out.append((off, min(step, total - off)))
        off += step
    return out


@functools.partial(
    pl.kernel,
    out_type=jax.ShapeDtypeStruct((NCHUNK_TOT, BC, 4 * L), jnp.float32),
    mesh=plsc.VectorSubcoreMesh(core_axis_name="c", subcore_axis_name="s"),
    compiler_params=pltpu.CompilerParams(needs_layout_passes=False,
                                         use_tc_tiling_on_sc=False),
    scratch_types=[
        pltpu.VMEM((_CTX_N,), jnp.int32),
        pltpu.VMEM((_PAIR_N,), jnp.int32),
        pltpu.VMEM((_CTX_N, EMBED), jnp.float32),
        pltpu.VMEM((_PAIR_N, EMBED), jnp.float32),
        pltpu.VMEM((BC, 4 * L), jnp.float32),
        pltpu.SemaphoreType.DMA,
    ],
)
def _sc_scores(w_in, w_out, ctx_idx, pair_idx, out,
               ctx_idx_v, pair_idx_v, ctx_rows, pair_rows, scores, sem):
    wid = lax.axis_index("s") * NC + lax.axis_index("c")

    def chunk_body(chunk, _):
        cidx = wid * NCHUNK + chunk
        base = cidx * BC

        pltpu.async_copy(ctx_idx.at[pl.ds(base * CTX, _CTX_N)], ctx_idx_v, sem)
        pltpu.async_copy(pair_idx.at[pl.ds(base * NPAIR, _PAIR_N)],
                         pair_idx_v, sem).wait()
        pltpu.make_async_copy(ctx_idx.at[pl.ds(base * CTX, _CTX_N)],
                              ctx_idx_v, sem).wait()

        copies = []
        for off, n in _dma_slices(_CTX_N):
            copies.append(pltpu.async_copy(
                w_in.at[ctx_idx_v.at[pl.ds(off, n)]],
                ctx_rows.at[pl.ds(off, n)], sem))
        for off, n in _dma_slices(_PAIR_N):
            copies.append(pltpu.async_copy(
                w_out.at[pair_idx_v.at[pl.ds(off, n)]],
                pair_rows.at[pl.ds(off, n)], sem))
        for cp in copies:
            cp.wait()

        lane_iota = jax.lax.iota(jnp.int32, L)

        def row_body(i, _):
            acc = tuple(jnp.zeros((L,), jnp.float32) for _ in range(4))
            for c in range(CTX):
                r = i * CTX + c
                acc = tuple(a + ctx_rows[r, pl.ds(j * L, L)]
                            for j, a in enumerate(acc))
            ctx = tuple(a * jnp.float32(1.0 / CTX) for a in acc)

            for g in range(4):
                vec = jnp.zeros((L,), jnp.float32)
                for t in range(L):
                    k = g * L + t
                    if k >= NPAIR:
                        break
                    r = i * NPAIR + k
                    p = ctx[0] * pair_rows[r, pl.ds(0, L)]
                    for j in range(1, 4):
                        p = p + ctx[j] * pair_rows[r, pl.ds(j * L, L)]
                    s = jnp.sum(p)
                    vec = jnp.where(lane_iota == t, s if k == 0 else -s, vec)
                scores[i, pl.ds(g * L, L)] = vec
            return 0

        lax.fori_loop(0, BC, row_body, 0)

        pltpu.async_copy(scores, out.at[cidx], sem).wait()
        return 0

    lax.fori_loop(0, NCHUNK, chunk_body, 0)


def _loss_body(x_ref, o_ref):
    x = x_ref[...]
    col = lax.broadcasted_iota(jnp.int32, x.shape, 1)
    y = jnp.where(col < NPAIR, jax.nn.log_sigmoid(x), 0.0)
    o_ref[0, 0] = jnp.sum(y) * jnp.float32(-1.0 / B)


_loss = pl.pallas_call(
    _loss_body,
    out_shape=jax.ShapeDtypeStruct((1, 1), jnp.float32),
    out_specs=pl.BlockSpec(memory_space=pltpu.SMEM),
)


def kernel(context_words, target_word, negative_words, W_in, W_out):
    ctx_idx = context_words.astype(jnp.int32).reshape(-1)
    pair_idx = jnp.concatenate(
        [target_word.astype(jnp.int32)[:, None],
         negative_words.astype(jnp.int32)], axis=1).reshape(-1)
    scores = _sc_scores(W_in, W_out, ctx_idx, pair_idx)
    x = scores.reshape(B, 4 * L)
    return _loss(x)[0, 0]


# speedup vs baseline: 6.5232x; 1.1623x over previous
import functools

import jax
import jax.numpy as jnp
from jax import lax
from jax.experimental import pallas as pl
from jax.experimental.pallas import tpu as pltpu
from jax.experimental.pallas import tpu_sc as plsc

VOCAB = 100000
EMBED = 64
B = 4096
CTX = 20
NEG = 50
NPAIR = NEG + 1

NC = 2
NS = 16
L = 16
NW = NC * NS
ROWS_W = B // NW
BC = 8
NCHUNK = ROWS_W // BC
NCHUNK_TOT = B // BC

_CTX_N = BC * CTX
_PAIR_N = BC * NPAIR
_IDX_CTX_W = ROWS_W * CTX
_IDX_PAIR_W = ROWS_W * NPAIR


def _dma_slices(total, step=128):
    out, off = [], 0
    while off < total:
        out.append((off, min(step, total - off)))
        off += step
    return out


@functools.partial(
    pl.kernel,
    out_type=jax.ShapeDtypeStruct((NCHUNK_TOT, BC, 4 * L), jnp.float32),
    mesh=plsc.VectorSubcoreMesh(core_axis_name="c", subcore_axis_name="s"),
    compiler_params=pltpu.CompilerParams(needs_layout_passes=False,
                                         use_tc_tiling_on_sc=False),
    scratch_types=[
        pltpu.VMEM((_IDX_CTX_W,), jnp.int32),
        pltpu.VMEM((_IDX_PAIR_W,), jnp.int32),
        pltpu.VMEM((_CTX_N, EMBED), jnp.float32),
        pltpu.VMEM((_CTX_N, EMBED), jnp.float32),
        pltpu.VMEM((_PAIR_N, EMBED), jnp.float32),
        pltpu.VMEM((_PAIR_N, EMBED), jnp.float32),
        pltpu.VMEM((BC, 4 * L), jnp.float32),
        pltpu.VMEM((BC, 4 * L), jnp.float32),
        pltpu.SemaphoreType.DMA,
        pltpu.SemaphoreType.DMA,
        pltpu.SemaphoreType.DMA,
        pltpu.SemaphoreType.DMA,
        pltpu.SemaphoreType.DMA,
    ],
)
def _sc_scores(w_in, w_out, ctx_idx, pair_idx, out,
               ctx_idx_v, pair_idx_v, ctx_rows0, ctx_rows1,
               pair_rows0, pair_rows1, scores0, scores1,
               isem, gsem0, gsem1, osem0, osem1):
    wid = lax.axis_index("s") * NC + lax.axis_index("c")
    wbase = wid * NCHUNK
    lane_iota = lax.iota(jnp.int32, L)

    bufs = ((ctx_rows0, pair_rows0, gsem0, scores0, osem0),
            (ctx_rows1, pair_rows1, gsem1, scores1, osem1))

    pltpu.async_copy(ctx_idx.at[pl.ds(wid * _IDX_CTX_W, _IDX_CTX_W)],
                     ctx_idx_v, isem)
    pltpu.async_copy(pair_idx.at[pl.ds(wid * _IDX_PAIR_W, _IDX_PAIR_W)],
                     pair_idx_v, isem).wait()
    pltpu.make_async_copy(ctx_idx.at[pl.ds(wid * _IDX_CTX_W, _IDX_CTX_W)],
                          ctx_idx_v, isem).wait()

    def gather_ops(g, p):
        cb, pb, sem, _, _ = bufs[p]
        co = g * _CTX_N
        po = g * _PAIR_N
        ops = []
        for off, n in _dma_slices(_CTX_N):
            ops.append((w_in.at[ctx_idx_v.at[pl.ds(co + off, n)]],
                        cb.at[pl.ds(off, n)], sem))
        for off, n in _dma_slices(_PAIR_N):
            ops.append((w_out.at[pair_idx_v.at[pl.ds(po + off, n)]],
                        pb.at[pl.ds(off, n)], sem))
        return ops

    def issue_gathers(g, p):
        for src, dst, sem in gather_ops(g, p):
            pltpu.async_copy(src, dst, sem)

    def wait_gathers(g, p):
        for src, dst, sem in gather_ops(g, p):
            pltpu.make_async_copy(src, dst, sem).wait()

    def compute(g, p):
        cb, pb, _, sc, _ = bufs[p]

        def row_body(i, _):
            acc = tuple(jnp.zeros((L,), jnp.float32) for _ in range(4))
            for c in range(CTX):
                r = i * CTX + c
                acc = tuple(a + cb[r, pl.ds(j * L, L)]
                            for j, a in enumerate(acc))
            ctx = tuple(a * jnp.float32(1.0 / CTX) for a in acc)

            for grp in range(4):
                vec = jnp.zeros((L,), jnp.float32)
                for t in range(L):
                    k = grp * L + t
                    if k >= NPAIR:
                        break
                    r = i * NPAIR + k
                    p_ = ctx[0] * pb[r, pl.ds(0, L)]
                    for j in range(1, 4):
                        p_ = p_ + ctx[j] * pb[r, pl.ds(j * L, L)]
                    s = jnp.sum(p_)
                    vec = jnp.where(lane_iota == t, s if k == 0 else -s, vec)
                sc[i, pl.ds(grp * L, L)] = vec
            return 0

        lax.fori_loop(0, BC, row_body, 0)

    def loop_body(gh, _):
        for p in (0, 1):
            g = 2 * gh + p
            sc, osem = bufs[p][3], bufs[p][4]

            @pl.when(g + 1 < NCHUNK)
            def _():
                issue_gathers(g + 1, 1 - p)

            wait_gathers(g, p)

            @pl.when(g >= 2)
            def _():
                pltpu.make_async_copy(sc, out.at[wbase + g - 2], osem).wait()

            compute(g, p)
            pltpu.async_copy(sc, out.at[wbase + g], osem)
        return 0

    issue_gathers(0, 0)
    lax.fori_loop(0, NCHUNK // 2, loop_body, 0)
    pltpu.make_async_copy(scores0, out.at[wbase + NCHUNK - 2], osem0).wait()
    pltpu.make_async_copy(scores1, out.at[wbase + NCHUNK - 1], osem1).wait()


def _loss_body(x_ref, o_ref):
    x = x_ref[...]
    col = lax.broadcasted_iota(jnp.int32, x.shape, 1)
    y = jnp.where(col < NPAIR, jax.nn.log_sigmoid(x), 0.0)
    o_ref[0, 0] = jnp.sum(y) * jnp.float32(-1.0 / B)


_loss = pl.pallas_call(
    _loss_body,
    out_shape=jax.ShapeDtypeStruct((1, 1), jnp.float32),
    out_specs=pl.BlockSpec(memory_space=pltpu.SMEM),
)


def kernel(context_words, target_word, negative_words, W_in, W_out):
    ctx_idx = context_words.astype(jnp.int32).reshape(-1)
    pair_idx = jnp.concatenate(
        [target_word.astype(jnp.int32)[:, None],
         negative_words.astype(jnp.int32)], axis=1).reshape(-1)
    scores = _sc_scores(W_in, W_out, ctx_idx, pair_idx)
    x = scores.reshape(B, 4 * L)
    return _loss(x)[0, 0]


# speedup vs baseline: 6.6476x; 1.0191x over previous
import functools

import jax
import jax.numpy as jnp
from jax import lax
from jax.experimental import pallas as pl
from jax.experimental.pallas import tpu as pltpu
from jax.experimental.pallas import tpu_sc as plsc

VOCAB = 100000
EMBED = 64
B = 4096
CTX = 20
NEG = 50
NPAIR = NEG + 1

NC = 2
NS = 16
L = 16
NW = NC * NS
ROWS_W = B // NW
BC = 8
NCHUNK = ROWS_W // BC
NCHUNK_TOT = B // BC

_CTX_N = BC * CTX
_PAIR_N = BC * NEG
_IDX_CTX_W = ROWS_W * CTX
_IDX_PAIR_W = ROWS_W * NEG


def _dma_slices(total, step=128):
    out, off = [], 0
    while off < total:
        out.append((off, min(step, total - off)))
        off += step
    return out


@functools.partial(
    pl.kernel,
    out_type=jax.ShapeDtypeStruct((NCHUNK_TOT, BC, 4 * L), jnp.float32),
    mesh=plsc.VectorSubcoreMesh(core_axis_name="c", subcore_axis_name="s"),
    compiler_params=pltpu.CompilerParams(needs_layout_passes=False,
                                         use_tc_tiling_on_sc=False),
    scratch_types=[
        pltpu.VMEM((_IDX_CTX_W,), jnp.int32),
        pltpu.VMEM((_IDX_PAIR_W,), jnp.int32),
        pltpu.VMEM((ROWS_W,), jnp.int32),
        pltpu.VMEM((ROWS_W, EMBED), jnp.float32),
        pltpu.VMEM((_CTX_N, EMBED), jnp.float32),
        pltpu.VMEM((_CTX_N, EMBED), jnp.float32),
        pltpu.VMEM((_PAIR_N, EMBED), jnp.float32),
        pltpu.VMEM((_PAIR_N, EMBED), jnp.float32),
        pltpu.VMEM((BC, 4 * L), jnp.float32),
        pltpu.VMEM((BC, 4 * L), jnp.float32),
        pltpu.SemaphoreType.DMA,
        pltpu.SemaphoreType.DMA,
        pltpu.SemaphoreType.DMA,
        pltpu.SemaphoreType.DMA,
        pltpu.SemaphoreType.DMA,
    ],
)
def _sc_scores(w_in, w_out, ctx_idx, pair_idx, tgt_idx, out,
               ctx_idx_v, pair_idx_v, tgt_idx_v, tgt_rows,
               ctx_rows0, ctx_rows1,
               pair_rows0, pair_rows1, scores0, scores1,
               isem, gsem0, gsem1, osem0, osem1):
    wid = lax.axis_index("s") * NC + lax.axis_index("c")
    wbase = wid * NCHUNK
    lane_iota = lax.iota(jnp.int32, L)

    bufs = ((ctx_rows0, pair_rows0, gsem0, scores0, osem0),
            (ctx_rows1, pair_rows1, gsem1, scores1, osem1))

    pltpu.async_copy(ctx_idx.at[pl.ds(wid * _IDX_CTX_W, _IDX_CTX_W)],
                     ctx_idx_v, isem)
    pltpu.async_copy(pair_idx.at[pl.ds(wid * _IDX_PAIR_W, _IDX_PAIR_W)],
                     pair_idx_v, isem)
    pltpu.async_copy(tgt_idx.at[pl.ds(wid * ROWS_W, ROWS_W)],
                     tgt_idx_v, isem).wait()
    pltpu.make_async_copy(ctx_idx.at[pl.ds(wid * _IDX_CTX_W, _IDX_CTX_W)],
                          ctx_idx_v, isem).wait()
    pltpu.make_async_copy(pair_idx.at[pl.ds(wid * _IDX_PAIR_W, _IDX_PAIR_W)],
                          pair_idx_v, isem).wait()
    for off, n in _dma_slices(ROWS_W):
        pltpu.async_copy(w_out.at[tgt_idx_v.at[pl.ds(off, n)]],
                         tgt_rows.at[pl.ds(off, n)], isem)
    for off, n in _dma_slices(ROWS_W):
        pltpu.make_async_copy(w_out.at[tgt_idx_v.at[pl.ds(off, n)]],
                              tgt_rows.at[pl.ds(off, n)], isem).wait()

    def gather_ops(g, p):
        cb, pb, sem, _, _ = bufs[p]
        co = g * _CTX_N
        po = g * _PAIR_N
        ops = []
        for off, n in _dma_slices(_CTX_N):
            ops.append((w_in.at[ctx_idx_v.at[pl.ds(co + off, n)]],
                        cb.at[pl.ds(off, n)], sem))
        for off, n in _dma_slices(_PAIR_N):
            ops.append((w_out.at[pair_idx_v.at[pl.ds(po + off, n)]],
                        pb.at[pl.ds(off, n)], sem))
        return ops

    def issue_gathers(g, p):
        for src, dst, sem in gather_ops(g, p):
            pltpu.async_copy(src, dst, sem)

    def wait_gathers(g, p):
        for src, dst, sem in gather_ops(g, p):
            pltpu.make_async_copy(src, dst, sem).wait()

    def compute(g, p):
        cb, pb, _, sc, _ = bufs[p]

        def row_body(i, _):
            acc = tuple(jnp.zeros((L,), jnp.float32) for _ in range(4))
            for c in range(CTX):
                r = i * CTX + c
                acc = tuple(a + cb[r, pl.ds(j * L, L)]
                            for j, a in enumerate(acc))
            ctx = tuple(a * jnp.float32(1.0 / CTX) for a in acc)

            for grp in range(4):
                vec = jnp.zeros((L,), jnp.float32)
                for t in range(L):
                    k = grp * L + t
                    if k >= NPAIR:
                        break
                    if k == 0:
                        r = g * BC + i
                        rb = tgt_rows
                    else:
                        r = i * NEG + (k - 1)
                        rb = pb
                    p_ = ctx[0] * rb[r, pl.ds(0, L)]
                    for j in range(1, 4):
                        p_ = p_ + ctx[j] * rb[r, pl.ds(j * L, L)]
                    s = jnp.sum(p_)
                    vec = jnp.where(lane_iota == t, s if k == 0 else -s, vec)
                sc[i, pl.ds(grp * L, L)] = vec
            return 0

        lax.fori_loop(0, BC, row_body, 0)

    def loop_body(gh, _):
        for p in (0, 1):
            g = 2 * gh + p
            sc, osem = bufs[p][3], bufs[p][4]

            @pl.when(g + 1 < NCHUNK)
            def _():
                issue_gathers(g + 1, 1 - p)

            wait_gathers(g, p)

            @pl.when(g >= 2)
            def _():
                pltpu.make_async_copy(sc, out.at[wbase + g - 2], osem).wait()

            compute(g, p)
            pltpu.async_copy(sc, out.at[wbase + g], osem)
        return 0

    issue_gathers(0, 0)
    lax.fori_loop(0, NCHUNK // 2, loop_body, 0)
    pltpu.make_async_copy(scores0, out.at[wbase + NCHUNK - 2], osem0).wait()
    pltpu.make_async_copy(scores1, out.at[wbase + NCHUNK - 1], osem1).wait()


def _loss_body(x_ref, o_ref):
    x = x_ref[...].reshape(B * 4 * L // 128, 128)
    col = lax.broadcasted_iota(jnp.int32, x.shape, 1)
    y = jnp.where(col % EMBED < NPAIR, jax.nn.log_sigmoid(x), 0.0)
    o_ref[0, 0] = jnp.sum(y) * jnp.float32(-1.0 / B)


_loss = pl.pallas_call(
    _loss_body,
    out_shape=jax.ShapeDtypeStruct((1, 1), jnp.float32),
    out_specs=pl.BlockSpec(memory_space=pltpu.SMEM),
)


def kernel(context_words, target_word, negative_words, W_in, W_out):
    ctx_idx = context_words.astype(jnp.int32).reshape(-1)
    neg_idx = negative_words.astype(jnp.int32).reshape(-1)
    tgt_idx = target_word.astype(jnp.int32)
    scores = _sc_scores(W_in, W_out, ctx_idx, neg_idx, tgt_idx)
    return _loss(scores.reshape(-1))[0, 0]


# speedup vs baseline: 6.7247x; 1.0116x over previous
import functools

import jax
import jax.numpy as jnp
from jax import lax
from jax.experimental import pallas as pl
from jax.experimental.pallas import tpu as pltpu
from jax.experimental.pallas import tpu_sc as plsc

VOCAB = 100000
EMBED = 64
B = 4096
CTX = 20
NEG = 50
NPAIR = NEG + 1

NC = 2
NS = 16
L = 16
NW = NC * NS
ROWS_W = B // NW
BC = 8
NCHUNK = ROWS_W // BC
NCHUNK_TOT = B // BC

_CTX_N = BC * CTX
_NEG_N = BC * NEG
_IDX_CTX_W = ROWS_W * CTX
_IDX_NEG_W = ROWS_W * NEG

_SC_PARAMS = pltpu.CompilerParams(needs_layout_passes=False,
                                  use_tc_tiling_on_sc=False)
_SC_MESH = dict(core_axis_name="c", subcore_axis_name="s")


def _dma_slices(total, step=128):
    out, off = [], 0
    while off < total:
        out.append((off, min(step, total - off)))
        off += step
    return out


@functools.partial(
    pl.kernel,
    out_type=jax.ShapeDtypeStruct((B, EMBED), jnp.float32),
    mesh=plsc.VectorSubcoreMesh(**_SC_MESH),
    compiler_params=_SC_PARAMS,
    scratch_types=[
        pltpu.VMEM((_IDX_CTX_W,), jnp.int32),
        pltpu.VMEM((_CTX_N, EMBED), jnp.float32),
        pltpu.VMEM((_CTX_N, EMBED), jnp.float32),
        pltpu.VMEM((BC, EMBED), jnp.float32),
        pltpu.VMEM((BC, EMBED), jnp.float32),
        pltpu.SemaphoreType.DMA,
        pltpu.SemaphoreType.DMA,
        pltpu.SemaphoreType.DMA,
        pltpu.SemaphoreType.DMA,
        pltpu.SemaphoreType.DMA,
    ],
)
def _sc_means(w_in, ctx_idx, means,
              ctx_idx_v, ctx_rows0, ctx_rows1, mean0, mean1,
              isem, gsem0, gsem1, osem0, osem1):
    wid = lax.axis_index("s") * NC + lax.axis_index("c")
    rbase = wid * ROWS_W

    bufs = ((ctx_rows0, gsem0, mean0, osem0),
            (ctx_rows1, gsem1, mean1, osem1))

    pltpu.async_copy(ctx_idx.at[pl.ds(wid * _IDX_CTX_W, _IDX_CTX_W)],
                     ctx_idx_v, isem).wait()

    def gather_ops(g, p):
        cb, sem = bufs[p][0], bufs[p][1]
        co = g * _CTX_N
        return [(w_in.at[ctx_idx_v.at[pl.ds(co + off, n)]],
                 cb.at[pl.ds(off, n)], sem)
                for off, n in _dma_slices(_CTX_N)]

    def compute(g, p):
        cb, mb = bufs[p][0], bufs[p][2]

        def row_body(i, _):
            acc = tuple(jnp.zeros((L,), jnp.float32) for _ in range(4))
            for c in range(CTX):
                r = i * CTX + c
                acc = tuple(a + cb[r, pl.ds(j * L, L)]
                            for j, a in enumerate(acc))
            for j, a in enumerate(acc):
                mb[i, pl.ds(j * L, L)] = a * jnp.float32(1.0 / CTX)
            return 0

        lax.fori_loop(0, BC, row_body, 0)

    def loop_body(gh, _):
        for p in (0, 1):
            g = 2 * gh + p
            mb, osem = bufs[p][2], bufs[p][3]

            @pl.when(g + 1 < NCHUNK)
            def _():
                for src, dst, sem in gather_ops(g + 1, 1 - p):
                    pltpu.async_copy(src, dst, sem)

            for src, dst, sem in gather_ops(g, p):
                pltpu.make_async_copy(src, dst, sem).wait()

            @pl.when(g >= 2)
            def _():
                pltpu.make_async_copy(
                    mb, means.at[pl.ds(rbase + (g - 2) * BC, BC)], osem).wait()

            compute(g, p)
            pltpu.async_copy(mb, means.at[pl.ds(rbase + g * BC, BC)], osem)
        return 0

    for src, dst, sem in gather_ops(0, 0):
        pltpu.async_copy(src, dst, sem)
    lax.fori_loop(0, NCHUNK // 2, loop_body, 0)
    pltpu.make_async_copy(
        mean0, means.at[pl.ds(rbase + (NCHUNK - 2) * BC, BC)], osem0).wait()
    pltpu.make_async_copy(
        mean1, means.at[pl.ds(rbase + (NCHUNK - 1) * BC, BC)], osem1).wait()


@functools.partial(
    pl.kernel,
    out_type=jax.ShapeDtypeStruct((NCHUNK_TOT, BC, 4 * L), jnp.float32),
    mesh=plsc.VectorSubcoreMesh(**_SC_MESH),
    compiler_params=_SC_PARAMS,
    scratch_types=[
        pltpu.VMEM((_IDX_NEG_W,), jnp.int32),
        pltpu.VMEM((ROWS_W,), jnp.int32),
        pltpu.VMEM((ROWS_W, EMBED), jnp.float32),
        pltpu.VMEM((_NEG_N, EMBED), jnp.float32),
        pltpu.VMEM((_NEG_N, EMBED), jnp.float32),
        pltpu.VMEM((BC, EMBED), jnp.float32),
        pltpu.VMEM((BC, EMBED), jnp.float32),
        pltpu.VMEM((BC, 4 * L), jnp.float32),
        pltpu.VMEM((BC, 4 * L), jnp.float32),
        pltpu.SemaphoreType.DMA,
        pltpu.SemaphoreType.DMA,
        pltpu.SemaphoreType.DMA,
        pltpu.SemaphoreType.DMA,
        pltpu.SemaphoreType.DMA,
    ],
)
def _sc_pair(w_out, neg_idx, tgt_idx, means, out,
             neg_idx_v, tgt_idx_v, tgt_rows,
             neg_rows0, neg_rows1, mean0, mean1, scores0, scores1,
             isem, gsem0, gsem1, osem0, osem1):
    wid = lax.axis_index("s") * NC + lax.axis_index("c")
    wbase = wid * NCHUNK
    rbase = wid * ROWS_W
    lane_iota = lax.iota(jnp.int32, L)

    bufs = ((neg_rows0, gsem0, mean0, scores0, osem0),
            (neg_rows1, gsem1, mean1, scores1, osem1))

    pltpu.async_copy(neg_idx.at[pl.ds(wid * _IDX_NEG_W, _IDX_NEG_W)],
                     neg_idx_v, isem)
    pltpu.async_copy(tgt_idx.at[pl.ds(rbase, ROWS_W)], tgt_idx_v, isem).wait()
    pltpu.make_async_copy(neg_idx.at[pl.ds(wid * _IDX_NEG_W, _IDX_NEG_W)],
                          neg_idx_v, isem).wait()
    for off, n in _dma_slices(ROWS_W):
        pltpu.async_copy(w_out.at[tgt_idx_v.at[pl.ds(off, n)]],
                         tgt_rows.at[pl.ds(off, n)], isem)

    def gather_ops(g, p):
        nb, sem, mb = bufs[p][0], bufs[p][1], bufs[p][2]
        no = g * _NEG_N
        ops = [(w_out.at[neg_idx_v.at[pl.ds(no + off, n)]],
                nb.at[pl.ds(off, n)], sem)
               for off, n in _dma_slices(_NEG_N)]
        ops.append((means.at[pl.ds(rbase + g * BC, BC)], mb, sem))
        return ops

    def compute(g, p):
        nb, mb, sc = bufs[p][0], bufs[p][2], bufs[p][3]

        def row_body(i, _):
            ctx = tuple(mb[i, pl.ds(j * L, L)] for j in range(4))

            for grp in range(4):
                vec = jnp.zeros((L,), jnp.float32)
                for t in range(L):
                    k = grp * L + t
                    if k >= NPAIR:
                        break
                    if k == 0:
                        r = g * BC + i
                        rb = tgt_rows
                    else:
                        r = i * NEG + (k - 1)
                        rb = nb
                    p_ = ctx[0] * rb[r, pl.ds(0, L)]
                    for j in range(1, 4):
                        p_ = p_ + ctx[j] * rb[r, pl.ds(j * L, L)]
                    s = jnp.sum(p_)
                    vec = jnp.where(lane_iota == t, s if k == 0 else -s, vec)
                sc[i, pl.ds(grp * L, L)] = vec
            return 0

        lax.fori_loop(0, BC, row_body, 0)

    def loop_body(gh, _):
        for p in (0, 1):
            g = 2 * gh + p
            sc, osem = bufs[p][3], bufs[p][4]

            @pl.when(g + 1 < NCHUNK)
            def _():
                for src, dst, sem in gather_ops(g + 1, 1 - p):
                    pltpu.async_copy(src, dst, sem)

            for src, dst, sem in gather_ops(g, p):
                pltpu.make_async_copy(src, dst, sem).wait()

            @pl.when(g >= 2)
            def _():
                pltpu.make_async_copy(sc, out.at[wbase + g - 2], osem).wait()

            compute(g, p)
            pltpu.async_copy(sc, out.at[wbase + g], osem)
        return 0

    for off, n in _dma_slices(ROWS_W):
        pltpu.make_async_copy(w_out.at[tgt_idx_v.at[pl.ds(off, n)]],
                              tgt_rows.at[pl.ds(off, n)], isem).wait()
    for src, dst, sem in gather_ops(0, 0):
        pltpu.async_copy(src, dst, sem)
    lax.fori_loop(0, NCHUNK // 2, loop_body, 0)
    pltpu.make_async_copy(scores0, out.at[wbase + NCHUNK - 2], osem0).wait()
    pltpu.make_async_copy(scores1, out.at[wbase + NCHUNK - 1], osem1).wait()


def _loss_body(x_ref, o_ref):
    x = x_ref[...].reshape(B * 4 * L // 128, 128)
    col = lax.broadcasted_iota(jnp.int32, x.shape, 1)
    y = jnp.where(col % EMBED < NPAIR, jax.nn.log_sigmoid(x), 0.0)
    o_ref[0, 0] = jnp.sum(y) * jnp.float32(-1.0 / B)


_loss = pl.pallas_call(
    _loss_body,
    out_shape=jax.ShapeDtypeStruct((1, 1), jnp.float32),
    out_specs=pl.BlockSpec(memory_space=pltpu.SMEM),
)


def kernel(context_words, target_word, negative_words, W_in, W_out):
    ctx_idx = context_words.astype(jnp.int32).reshape(-1)
    neg_idx = negative_words.astype(jnp.int32).reshape(-1)
    tgt_idx = target_word.astype(jnp.int32)
    means = _sc_means(W_in, ctx_idx)
    scores = _sc_pair(W_out, neg_idx, tgt_idx, means)
    return _loss(scores.reshape(-1))[0, 0]
